# baseline (speedup 1.0000x reference)
import functools

import jax
import jax.numpy as jnp
from jax import lax
from jax.experimental import pallas as pl
from jax.experimental.pallas import tpu as pltpu
from jax.experimental.pallas import tpu_sc as plsc

N_ENTITY = 1000000
N_RELATION = 1000
BATCH = 16384
DIM = 64

NC = 2
NS = 16
L = 16
NW = NC * NS
B_PER_W = BATCH // NW
IDX_CHUNK = 128
N_CHUNK = B_PER_W // IDX_CHUNK


def _body(head_hbm, tail_hbm, rel_hbm, emb_e_hbm, emb_r_hbm, out_hbm,
          hidx_v, tidx_v, ridx_v, h_v, t_v, r_v, s_v, out_v, sem):
    wid = lax.axis_index("s") * NC + lax.axis_index("c")
    base = wid * B_PER_W

    pltpu.sync_copy(head_hbm.at[pl.ds(base, B_PER_W)], hidx_v)
    pltpu.sync_copy(tail_hbm.at[pl.ds(base, B_PER_W)], tidx_v)
    pltpu.sync_copy(rel_hbm.at[pl.ds(base, B_PER_W)], ridx_v)

    copies = []
    for j in range(N_CHUNK):
        sl = pl.ds(j * IDX_CHUNK, IDX_CHUNK)
        copies.append(pltpu.async_copy(emb_e_hbm.at[hidx_v.at[sl]], h_v.at[sl], sem))
        copies.append(pltpu.async_copy(emb_e_hbm.at[tidx_v.at[sl]], t_v.at[sl], sem))
        copies.append(pltpu.async_copy(emb_r_hbm.at[ridx_v.at[sl]], r_v.at[sl], sem))
    for c in copies:
        c.wait()

    def row_fold(row, carry):
        acc = jnp.zeros((L,), jnp.float32)
        for k in range(DIM // L):
            sl = pl.ds(k * L, L)
            acc = acc + h_v[row, sl] * t_v[row, sl] * r_v[row, sl]
        s_v[pl.ds(row * L, L)] = acc
        return carry

    lax.fori_loop(0, B_PER_W, row_fold, 0)

    lane = lax.broadcasted_iota(jnp.int32, (L,), 0)

    def group(g, carry):
        base16 = g * (L * L) + lane * L
        acc = plsc.load_gather(s_v, [base16])
        for j in range(1, L):
            acc = acc + plsc.load_gather(s_v, [base16 + j])
        out_v[pl.ds(g * L, L)] = acc
        return carry

    lax.fori_loop(0, B_PER_W // L, group, 0)

    pltpu.sync_copy(out_v, out_hbm.at[pl.ds(base, B_PER_W)])


@jax.jit
def kernel(head, tail, relation, emb_E, emb_R):
    mesh = plsc.VectorSubcoreMesh(
        core_axis_name="c", subcore_axis_name="s", num_cores=NC, num_subcores=NS
    )
    run = pl.kernel(
        _body,
        out_type=jax.ShapeDtypeStruct((BATCH,), jnp.float32),
        mesh=mesh,
        compiler_params=pltpu.CompilerParams(
            needs_layout_passes=False, use_tc_tiling_on_sc=False
        ),
        scratch_types=[
            pltpu.VMEM((B_PER_W,), jnp.int32),
            pltpu.VMEM((B_PER_W,), jnp.int32),
            pltpu.VMEM((B_PER_W,), jnp.int32),
            pltpu.VMEM((B_PER_W, DIM), jnp.float32),
            pltpu.VMEM((B_PER_W, DIM), jnp.float32),
            pltpu.VMEM((B_PER_W, DIM), jnp.float32),
            pltpu.VMEM((B_PER_W * L,), jnp.float32),
            pltpu.VMEM((B_PER_W,), jnp.float32),
            pltpu.SemaphoreType.DMA,
        ],
    )
    return run(head.astype(jnp.int32), tail.astype(jnp.int32),
               relation.astype(jnp.int32), emb_E, emb_R)
